# baseline (speedup 1.0000x reference)
import functools
import jax
import jax.numpy as jnp
from jax import lax
from jax.experimental import pallas as pl
from jax.experimental.pallas import tpu as pltpu
from jax.experimental.pallas import tpu_sc as plsc

_B = 4
_T = 8192
_D = 1024
_NC = 2
_NS = 16
_NW = _NC * _NS
_TPW = _T // _NW
_R = 8
_NCHUNK = _TPW // _R


@functools.partial(
    pl.kernel,
    mesh=plsc.VectorSubcoreMesh(core_axis_name="c", subcore_axis_name="s"),
    out_type=jax.ShapeDtypeStruct((_B, _T, _D), jnp.float32),
    scratch_types=[
        pltpu.VMEM((_R, _D), jnp.float32),
        pltpu.VMEM((_R, _D), jnp.float32),
    ],
)
def _sc_add(x_hbm, pe_hbm, out_hbm, pe_buf, x_buf):
    wid = lax.axis_index("s") * _NC + lax.axis_index("c")
    t0 = wid * _TPW

    def chunk_body(i, carry):
        t = t0 + i * _R
        pltpu.sync_copy(pe_hbm.at[pl.ds(t, _R), :], pe_buf)
        for b in range(_B):
            pltpu.sync_copy(x_hbm.at[b, pl.ds(t, _R), :], x_buf)

            def col(j, c2):
                r = j // (_D // 16)
                sl = pl.ds((j % (_D // 16)) * 16, 16)
                x_buf[r, sl] = x_buf[r, sl] + pe_buf[r, sl]
                return c2

            lax.fori_loop(0, _R * (_D // 16), col, 0)
            pltpu.sync_copy(x_buf, out_hbm.at[b, pl.ds(t, _R), :])
        return carry

    lax.fori_loop(0, _NCHUNK, chunk_body, 0)


def kernel(x, pe_weight):
    return _sc_add(x, pe_weight)


# speedup vs baseline: 3.7159x; 3.7159x over previous
import functools
import jax
import jax.numpy as jnp
from jax import lax
from jax.experimental import pallas as pl
from jax.experimental.pallas import tpu as pltpu
from jax.experimental.pallas import tpu_sc as plsc

_B = 4
_T = 8192
_D = 1024
_NC = 2
_NS = 16
_NW = _NC * _NS
_TPW = _T // _NW
_R = 8
_NCHUNK = _TPW // _R
_NV = _D // 16


@functools.partial(
    pl.kernel,
    mesh=plsc.VectorSubcoreMesh(core_axis_name="c", subcore_axis_name="s"),
    out_type=jax.ShapeDtypeStruct((_B, _T, _D), jnp.float32),
    scratch_types=[
        pltpu.VMEM((_B, _R, _D), jnp.float32),
        pltpu.VMEM((_B, _R, _D), jnp.float32),
        pltpu.VMEM((_R, _D), jnp.float32),
        pltpu.VMEM((_R, _D), jnp.float32),
        pltpu.SemaphoreType.DMA,
        pltpu.SemaphoreType.DMA,
        pltpu.SemaphoreType.DMA,
        pltpu.SemaphoreType.DMA,
    ],
)
def _sc_add(x_hbm, pe_hbm, out_hbm, xb0, xb1, pb0, pb1, in0, in1, out0, out1):
    wid = lax.axis_index("s") * _NC + lax.axis_index("c")
    t0 = wid * _TPW
    xb = (xb0, xb1)
    pb = (pb0, pb1)
    in_sem = (in0, in1)
    out_sem = (out0, out1)

    pending_in = [[], []]
    pending_out = [[], []]

    def issue_inputs(i):
        slot = i % 2
        t = t0 + i * _R
        hs = [pltpu.async_copy(pe_hbm.at[pl.ds(t, _R), :], pb[slot], in_sem[slot])]
        for b in range(_B):
            hs.append(
                pltpu.async_copy(
                    x_hbm.at[b, pl.ds(t, _R), :], xb[slot].at[b], in_sem[slot]
                )
            )
        pending_in[slot] = hs

    def compute_and_writeback(i):
        slot = i % 2
        t = t0 + i * _R
        for h in pending_in[slot]:
            h.wait()
        pending_in[slot] = []

        def col(j, c2):
            r = j // _NV
            sl = pl.ds((j % _NV) * 16, 16)
            pv = pb[slot][r, sl]
            for b in range(_B):
                xb[slot][b, r, sl] = xb[slot][b, r, sl] + pv
            return c2

        lax.fori_loop(0, _R * _NV, col, 0)
        hs = []
        for b in range(_B):
            hs.append(
                pltpu.async_copy(
                    xb[slot].at[b], out_hbm.at[b, pl.ds(t, _R), :], out_sem[slot]
                )
            )
        pending_out[slot] = hs

    for i in range(_NCHUNK + 1):
        if i < _NCHUNK:
            slot = i % 2
            for h in pending_out[slot]:
                h.wait()
            pending_out[slot] = []
            issue_inputs(i)
        if i > 0:
            compute_and_writeback(i - 1)
    for slot in (0, 1):
        for h in pending_out[slot]:
            h.wait()


def kernel(x, pe_weight):
    return _sc_add(x, pe_weight)
